# baseline (speedup 1.0000x reference)
import functools

import jax
import jax.numpy as jnp
from jax import lax
from jax.experimental import pallas as pl
from jax.experimental.pallas import tpu as pltpu
from jax.experimental.pallas import tpu_sc as plsc

RES_ = 128
C_ = 8
N_ = 500000
V_ = RES_ * RES_ * RES_
VH_ = V_ // 2
VQ_ = V_ // 4
NPAD_ = 512000
DUM_ = 8192
PTS_PER_TILE_ = NPAD_ // 32
PTS_PER_SC_TILE_ = NPAD_ // 16
ZCH_ = (VQ_ + DUM_) // 16
ZBUF_ = ZCH_ // 8
CAPH_ = 4800
CAP_ = 2 * CAPH_
CAPP_ = CAP_ + 64


def _mesh():
    return plsc.VectorSubcoreMesh(
        core_axis_name="c", subcore_axis_name="s", num_cores=2, num_subcores=16
    )


def _idx_prep_body(crd_hbm, idx_hbm, cbuf, i0):
    w = lax.axis_index("c") * 16 + lax.axis_index("s")
    base = w * PTS_PER_TILE_
    pltpu.sync_copy(crd_hbm.at[pl.ds(base * 4, PTS_PER_TILE_ * 4)], cbuf)
    lane = lax.iota(jnp.int32, 16)

    def body(i, _):
        col = (i * 16 + lane) * 4
        x = plsc.load_gather(cbuf, [col])
        y = plsc.load_gather(cbuf, [col + 1])
        z = plsc.load_gather(cbuf, [col + 2])
        xi = jnp.clip((x * float(RES_)).astype(jnp.int32), 0, RES_ - 1)
        yi = jnp.clip((y * float(RES_)).astype(jnp.int32), 0, RES_ - 1)
        zi = jnp.clip((z * float(RES_)).astype(jnp.int32), 0, RES_ - 1)
        flat = (xi * RES_ + yi) * RES_ + zi
        g = base + i * 16 + lane
        i0[pl.ds(i * 16, 16)] = jnp.where(g < N_, flat, V_)
        return 0

    lax.fori_loop(0, PTS_PER_TILE_ // 16, body, 0)
    pltpu.sync_copy(i0, idx_hbm.at[pl.ds(base, PTS_PER_TILE_)])


def _scatter_body(h, idx_hbm, vals_hbm, out_hbm, idx_v, val_v, zbuf, cidx, cperm, cval, acc):
    c = lax.axis_index("c")
    s = lax.axis_index("s")
    lane = lax.iota(jnp.int32, 16)
    half = PTS_PER_SC_TILE_ // 2

    def zb(i, _):
        zbuf[pl.ds(i * 16, 16)] = jnp.zeros((16,), jnp.float32)
        return 0

    lax.fori_loop(0, ZBUF_ // 16, zb, 0)
    for r in range(8):
        pltpu.sync_copy(zbuf, acc.at[pl.ds(s * ZCH_ + r * ZBUF_, ZBUF_)])
    plsc.subcore_barrier()
    pltpu.sync_copy(
        idx_hbm.at[pl.ds(s * PTS_PER_SC_TILE_, PTS_PER_SC_TILE_)], idx_v
    )
    if True:
        k = 2 * h + c

        def pre(i, _):
            for u in range(4):
                j = (i * 4 + u) * 16
                cidx[pl.ds(j, 16)] = VQ_ + ((j + lane) & (DUM_ - 1))
                cperm[pl.ds(j, 16)] = jnp.zeros((16,), jnp.int32)
            return 0

        lax.fori_loop(0, CAPP_ // 64, pre, 0)

        last = jnp.zeros((16,), jnp.int32) + 15

        def prefix(x):
            for d in (1, 2, 4, 8):
                sh = x.at[jnp.maximum(lane - d, 0)].get(mode="promise_in_bounds")
                x = x + jnp.where(lane >= d, sh, 0)
            return x

        def compress(i, offs):
            offa, offb = offs
            dum = VQ_ + ((i * 16 + lane) & (DUM_ - 1))
            da = idx_v[pl.ds(i * 16, 16)] - k * VQ_
            db = idx_v[pl.ds(half + i * 16, 16)] - k * VQ_
            ma = lax.bitcast_convert_type(da, jnp.uint32) < jnp.uint32(VQ_)
            mb = lax.bitcast_convert_type(db, jnp.uint32) < jnp.uint32(VQ_)
            csa = prefix(jnp.where(ma, 1, 0))
            csb = prefix(jnp.where(mb, 1, 0))
            pa = jnp.where(ma, offa + csa - 1, CAP_ + lane)
            pb = jnp.where(mb, offb + csb - 1, CAP_ + lane)
            plsc.store_scatter(cidx, [pa], jnp.where(ma, da, dum))
            plsc.store_scatter(cperm, [pa], i * 16 + lane)
            plsc.store_scatter(cidx, [pb], jnp.where(mb, db, dum))
            plsc.store_scatter(cperm, [pb], half + i * 16 + lane)
            na = csa.at[last].get(mode="promise_in_bounds")
            nb = csb.at[last].get(mode="promise_in_bounds")
            return (
                jnp.minimum(offa + na, CAPH_ - 16),
                jnp.minimum(offb + nb, CAP_ - 16),
            )

        zero16 = jnp.zeros((16,), jnp.int32)
        lax.fori_loop(0, half // 16, compress, (zero16, zero16 + CAPH_))

        for q in (8, 0, 1, 2, 3, 4, 5, 6, 7):
            if q == 8:
                def ones_fill(i, _):
                    for u in range(4):
                        j = (i * 4 + u) * 16
                        cval[pl.ds(j, 16)] = jnp.zeros((16,), jnp.float32) + 1.0
                    return 0

                lax.fori_loop(0, CAPP_ // 64, ones_fill, 0)
            else:
                pltpu.sync_copy(
                    vals_hbm.at[
                        pl.ds(q * NPAD_ + s * PTS_PER_SC_TILE_, PTS_PER_SC_TILE_)
                    ],
                    val_v,
                )

                def gat(i, _):
                    for u in range(4):
                        j = (i * 4 + u) * 16
                        p = cperm[pl.ds(j, 16)]
                        cval[pl.ds(j, 16)] = plsc.load_gather(val_v, [p])
                    return 0

                lax.fori_loop(0, CAPP_ // 64, gat, 0)
            pltpu.sync_copy(cval, acc.at[cidx], add=True)
            plsc.subcore_barrier()
            off = VQ_ // 16
            pltpu.sync_copy(
                acc.at[pl.ds(s * off, off)],
                out_hbm.at[pl.ds(q * VH_ + c * VQ_ + s * off, off)],
            )
            if q != 7:
                for r in range(8):
                    pltpu.sync_copy(zbuf, acc.at[pl.ds(s * ZCH_ + r * ZBUF_, ZBUF_)])
            plsc.subcore_barrier()


def _project_body(first, d_ref, *refs):
    if first:
        p0_ref, p1_ref, p2_ref = refs
    else:
        p0a_ref, p0_ref, p1_ref, p2_ref = refs
    i = pl.program_id(0)
    blk = d_ref[...]
    inv = 1.0 / jnp.maximum(blk[8], 1.0)
    mean = blk[0:8] * inv[None]
    p1_ref[...] = jnp.max(mean, axis=2)
    p2_ref[...] = jnp.max(mean, axis=3)
    ph = jnp.max(mean, axis=1)

    @pl.when(i == 0)
    def _():
        if first:
            p0_ref[...] = ph
        else:
            p0_ref[...] = jnp.maximum(p0a_ref[...], ph)

    @pl.when(i != 0)
    def _():
        p0_ref[...] = jnp.maximum(p0_ref[...], ph)


def kernel(coords, feats, box_class):
    f32 = jnp.float32
    crd = jnp.pad(coords.astype(f32).reshape(-1), (0, (NPAD_ - N_) * 4))
    vals = jnp.pad(feats.astype(f32).T, ((0, 0), (0, NPAD_ - N_))).reshape(-1)

    idx2 = pl.kernel(
        _idx_prep_body,
        out_type=jax.ShapeDtypeStruct((NPAD_,), jnp.int32),
        mesh=_mesh(),
        compiler_params=pltpu.CompilerParams(needs_layout_passes=False),
        scratch_types=[
            pltpu.VMEM((PTS_PER_TILE_ * 4,), f32),
            pltpu.VMEM((PTS_PER_TILE_,), jnp.int32),
        ],
    )(crd)

    def scatter_half(h):
        return pl.kernel(
            functools.partial(_scatter_body, h),
            out_type=jax.ShapeDtypeStruct((9 * VH_,), f32),
            mesh=_mesh(),
            compiler_params=pltpu.CompilerParams(needs_layout_passes=False),
            scratch_types=[
                pltpu.VMEM((PTS_PER_SC_TILE_,), jnp.int32),
                pltpu.VMEM((PTS_PER_SC_TILE_,), f32),
                pltpu.VMEM((ZBUF_,), f32),
                pltpu.VMEM((CAPP_,), jnp.int32),
                pltpu.VMEM((CAPP_,), jnp.int32),
                pltpu.VMEM((CAPP_,), f32),
                pltpu.VMEM_SHARED((VQ_ + DUM_,), f32),
            ],
        )(idx2, vals)

    dense0 = scatter_half(0)
    dense1 = scatter_half(1)

    grid = 8
    hs = RES_ // 2 // grid
    HH = RES_ // 2

    def project(first, d, p0a):
        d4 = d.reshape(9, HH, RES_, RES_)
        in_specs = [pl.BlockSpec((9, hs, RES_, RES_), lambda i: (0, i, 0, 0))]
        args = [d4]
        if not first:
            in_specs.append(pl.BlockSpec((C_, RES_, RES_), lambda i: (0, 0, 0)))
            args.append(p0a)
        return pl.pallas_call(
            functools.partial(_project_body, first),
            grid=(grid,),
            in_specs=in_specs,
            out_specs=[
                pl.BlockSpec((C_, RES_, RES_), lambda i: (0, 0, 0)),
                pl.BlockSpec((C_, hs, RES_), lambda i: (0, i, 0)),
                pl.BlockSpec((C_, hs, RES_), lambda i: (0, i, 0)),
            ],
            out_shape=[
                jax.ShapeDtypeStruct((C_, RES_, RES_), f32),
                jax.ShapeDtypeStruct((C_, HH, RES_), f32),
                jax.ShapeDtypeStruct((C_, HH, RES_), f32),
            ],
        )(*args)

    p0a, p1a, p2a = project(True, dense0, None)
    p0, p1b, p2b = project(False, dense1, p0a)

    p1 = jnp.concatenate([p1a, p1b], axis=1)
    p2 = jnp.concatenate([p2a, p2b], axis=1)
    view_mask = jnp.stack([p0, p1, p2], axis=0)
    img_class = jnp.tile(box_class, 3)
    return view_mask, img_class
